# baseline (speedup 1.0000x reference)
import functools
import math

import jax
import jax.numpy as jnp
from jax import lax
from jax.experimental import pallas as pl
from jax.experimental.pallas import tpu as pltpu
from jax.experimental.pallas import tpu_sc as plsc

F32 = jnp.float32
_EPS = 1e-5
_INV_SQRT2 = 0.7071067811865476


def _gelu(x):
    return 0.5 * x * (1.0 + lax.erf(x * _INV_SQRT2))


def _ln(x, g, b):
    m = jnp.mean(x, -1, keepdims=True)
    xc = x - m
    v = jnp.mean(xc * xc, -1, keepdims=True)
    return xc / jnp.sqrt(v + _EPS) * g + b


def _mm(a, b):
    return lax.dot_general(a, b, (((1,), (0,)), ((), ())),
                           preferred_element_type=F32)


def _head_body(pc_ref, W_ref, b_ref, g_ref, be_ref, Wf_ref, Wr_ref,
               feat_ref, u_ref):
    pc = pc_ref[0]
    f = _gelu(_ln(_mm(pc, W_ref[...]) + b_ref[...], g_ref[...], be_ref[...]))
    feat_ref[0] = f
    xyz = pc[:, :3]
    u_ref[0] = _mm(f, Wf_ref[...]) + _mm(xyz, Wr_ref[...])


def _head(pc, W, b, g, be, Wf, Wr_scaled, TN=1024):
    B, N, _ = pc.shape
    H = Wf.shape[1]
    grid = (B, N // TN)
    return pl.pallas_call(
        _head_body,
        grid=grid,
        in_specs=[
            pl.BlockSpec((1, TN, 6), lambda bb, t: (bb, t, 0)),
            pl.BlockSpec(W.shape, lambda bb, t: (0, 0)),
            pl.BlockSpec(b.shape, lambda bb, t: (0, 0)),
            pl.BlockSpec(g.shape, lambda bb, t: (0, 0)),
            pl.BlockSpec(be.shape, lambda bb, t: (0, 0)),
            pl.BlockSpec(Wf.shape, lambda bb, t: (0, 0)),
            pl.BlockSpec(Wr_scaled.shape, lambda bb, t: (0, 0)),
        ],
        out_specs=[
            pl.BlockSpec((1, TN, 64), lambda bb, t: (bb, t, 0)),
            pl.BlockSpec((1, TN, H), lambda bb, t: (bb, t, 0)),
        ],
        out_shape=[
            jax.ShapeDtypeStruct((B, N, 64), F32),
            jax.ShapeDtypeStruct((B, N, H), F32),
        ],
    )(pc, W, b, g, be, Wf, Wr_scaled)


_CH = 16


def _dc_body(pts_ref, cxt_ref, out_ref, *, N, TML):
    P = pts_ref[0]
    C = cxt_ref[0]
    pn2 = jnp.sum(P * P, axis=1, keepdims=True)
    cn2 = jnp.sum(C * C, axis=0, keepdims=True)
    dot = lax.dot_general(P, C, (((1,), (0,)), ((), ())),
                          preferred_element_type=F32,
                          precision=lax.Precision.HIGHEST)
    d = pn2 + cn2 - 2.0 * dot
    dc = jnp.min(d.reshape(N // _CH, _CH, TML), axis=1)
    out_ref[0] = dc


def _dc_chunkmin(xyz, pxt, M, N, TML=128):
    B = xyz.shape[0]
    NC = N // _CH
    body = functools.partial(_dc_body, N=N, TML=TML)
    return pl.pallas_call(
        body,
        grid=(B, M // TML),
        in_specs=[
            pl.BlockSpec((1, N, 3), lambda bb, t: (bb, 0, 0)),
            pl.BlockSpec((1, 3, TML), lambda bb, t: (bb, 0, t)),
        ],
        out_specs=pl.BlockSpec((1, NC, TML), lambda bb, t: (bb, 0, t)),
        out_shape=jax.ShapeDtypeStruct((B, NC, M), F32),
    )(xyz, pxt)


def _sc_ballquery(dc_flat, xp, yp, zp, B, M, N, NS, K, radius):
    NC = N // _CH
    r2 = radius * radius
    r2m = r2 * 1.0001 + 1e-5
    NW = 32
    RPW = (B * M) // NW
    assert (B * M) % NW == 0 and M % RPW == 0
    GMAX = NC // 16
    CAP = N + 16
    BIGF = jnp.float32(1e30)
    BIGI = jnp.int32(2 ** 30)

    mesh = plsc.VectorSubcoreMesh(core_axis_name="c", subcore_axis_name="s")

    @functools.partial(
        pl.kernel,
        mesh=mesh,
        out_type=jax.ShapeDtypeStruct((B * M * K,), jnp.int32),
        compiler_params=pltpu.CompilerParams(needs_layout_passes=False),
        scratch_types=[
            pltpu.VMEM((RPW * NC,), F32),
            pltpu.VMEM((N,), F32),
            pltpu.VMEM((N,), F32),
            pltpu.VMEM((N,), F32),
            pltpu.VMEM((2 * (NC + 16),), jnp.int32),
            pltpu.VMEM((2 * CAP,), jnp.int32),
            pltpu.VMEM((2 * CAP,), F32),
            pltpu.VMEM((RPW * K + 16,), jnp.int32),
            pltpu.SemaphoreType.DMA,
        ],
    )
    def kq(dc_hbm, xp_hbm, yp_hbm, zp_hbm, out_hbm,
           dcv, xv, yv, zv, ccv, piv, pdv, outv, sem):
        wid = lax.axis_index("s") * 2 + lax.axis_index("c")
        row0 = wid * RPW
        b = row0 // M
        cbase = row0 - b * M
        pltpu.sync_copy(dc_hbm.at[pl.ds(row0 * NC, RPW * NC)], dcv)
        pltpu.sync_copy(xp_hbm.at[pl.ds(b * NS, N)], xv)
        pltpu.sync_copy(yp_hbm.at[pl.ds(b * NS, N)], yv)
        pltpu.sync_copy(zp_hbm.at[pl.ds(b * NS, N)], zv)
        lanes = lax.broadcasted_iota(jnp.int32, (16,), 0)

        @plsc.parallel_loop(0, RPW, 1, unroll=2)
        def row_body(r):
            par = r % 2
            coff = par * (NC + 16)
            poff = par * CAP
            cloc = cbase + r
            csp = jnp.full((16,), cloc, jnp.int32)
            cx = plsc.load_gather(xv, [csp])
            cy = plsc.load_gather(yv, [csp])
            cz = plsc.load_gather(zv, [csp])

            def p1(c, carry):
                nccv, vm, im = carry
                v = dcv[pl.ds(r * NC + c * 16, 16)]
                ids = lanes + c * 16
                mask = v <= r2m
                cnt = plsc.all_reduce_population_count(mask)
                pos = nccv + plsc.cumsum(mask.astype(jnp.int32)) - 1
                plsc.store_scatter(ccv, [coff + pos], ids, mask=mask)
                im = jnp.where(v < vm, ids, im)
                vm = jnp.minimum(vm, v)
                return nccv + cnt, vm, im

            nccv, vm, im = lax.fori_loop(
                0, GMAX, p1,
                (jnp.zeros((16,), jnp.int32), jnp.full((16,), BIGF),
                 jnp.zeros((16,), jnp.int32)))
            ncc = jnp.max(nccv)
            mch = jnp.min(vm)
            amc = jnp.min(jnp.where(vm == mch, im, BIGI))
            plsc.store_scatter(ccv, [coff + lanes],
                               jnp.full((16,), amc, jnp.int32),
                               mask=(lanes == 0) & (ncc == 0))
            ncc = jnp.maximum(ncc, 1)

            def p2(g, carry):
                npv, tmv, tav = carry
                gids = plsc.load_gather(ccv, [coff + lanes + g * 16])
                valid = (lanes + g * 16) < ncc
                gids = jnp.where(valid, gids, 0)
                for j in range(_CH):
                    pt = gids * _CH + j
                    dx = plsc.load_gather(xv, [pt]) - cx
                    dy = plsc.load_gather(yv, [pt]) - cy
                    dz = plsc.load_gather(zv, [pt]) - cz
                    d2 = dx * dx + dy * dy + dz * dz
                    d2 = jnp.where(valid, d2, BIGF)
                    inr = d2 <= r2
                    pos = npv + plsc.cumsum(inr.astype(jnp.int32)) - 1
                    plsc.store_scatter(piv, [poff + pos], pt, mask=inr)
                    plsc.store_scatter(pdv, [poff + pos], d2, mask=inr)
                    npv = npv + plsc.all_reduce_population_count(inr)
                    tav = jnp.where(d2 < tmv, pt, tav)
                    tmv = jnp.minimum(tmv, d2)
                return npv, tmv, tav

            npv, tmv, tav = lax.fori_loop(
                0, (ncc + 15) // 16, p2,
                (jnp.zeros((16,), jnp.int32), jnp.full((16,), BIGF),
                 jnp.zeros((16,), jnp.int32)))
            np_ = jnp.max(npv)
            tmn = jnp.min(tmv)
            pad = jnp.min(jnp.where(tmv == tmn, tav, BIGI))
            padv = jnp.full((16,), pad, jnp.int32)

            off = r * K
            for s in range(0, K, 16):
                plsc.store_scatter(outv, [off + s + lanes], padv, mask=lanes < (K - s))

            @pl.when(np_ <= K)
            def _small():
                def emit(g, _c):
                    pos = g * 16 + lanes
                    vals = plsc.load_gather(piv, [poff + jnp.where(pos < np_, pos, 0)])
                    plsc.store_scatter(outv, [off + pos], vals, mask=pos < np_)
                    return _c
                lax.fori_loop(0, (np_ + 15) // 16, emit, 0)

            @pl.when(np_ > K)
            def _big():
                gsel = (np_ + 15) // 16

                def pick(kk, _c):
                    def scan(g, carry):
                        mv, pv = carry
                        pos = g * 16 + lanes
                        v = plsc.load_gather(pdv, [poff + jnp.where(pos < np_, pos, 0)])
                        v = jnp.where(pos < np_, v, BIGF)
                        pv = jnp.where(v < mv, pos, pv)
                        mv = jnp.minimum(mv, v)
                        return mv, pv
                    mv, pv = lax.fori_loop(
                        0, gsel, scan,
                        (jnp.full((16,), BIGF), jnp.zeros((16,), jnp.int32)))
                    mn = jnp.min(mv)
                    sel = jnp.min(jnp.where(mv == mn, pv, BIGI))
                    selv = jnp.full((16,), sel, jnp.int32)
                    val = plsc.load_gather(piv, [poff + selv])
                    plsc.store_scatter(outv, [off + kk + lanes], val, mask=lanes == 0)
                    plsc.store_scatter(pdv, [poff + selv], jnp.full((16,), BIGF),
                                       mask=lanes == 0)
                    return _c
                lax.fori_loop(0, K, pick, 0)


        pltpu.sync_copy(outv.at[pl.ds(0, RPW * K)],
                        out_hbm.at[pl.ds(row0 * K, RPW * K)])

    return kq(dc_flat, xp, yp, zp)


def _ball_query(xyz, pxt, xp, yp, zp, M, N, K, radius):
    B, NS = xyz.shape[0], xyz.shape[1]
    dct = _dc_chunkmin(xyz[:, :N], pxt[:, :, :M], M, N)
    dc_flat = jnp.transpose(dct, (0, 2, 1)).reshape(-1)
    idx = _sc_ballquery(dc_flat, xp, yp, zp, B, M, N, NS, K, radius)
    return idx.reshape(B, M, K)


def _sc_gather(table, idx):
    R = idx.shape[0]
    H = table.shape[1]
    NW = 32
    per_w = R // NW
    CH = 128
    n_chunks = per_w // CH
    assert per_w % CH == 0 and R % NW == 0

    mesh = plsc.VectorSubcoreMesh(core_axis_name="c", subcore_axis_name="s")

    @functools.partial(
        pl.kernel,
        mesh=mesh,
        out_type=jax.ShapeDtypeStruct((R, H), F32),
        scratch_types=[
            pltpu.VMEM((CH,), jnp.int32),
            pltpu.VMEM((CH, H), F32),
            pltpu.SemaphoreType.DMA,
        ],
    )
    def k(table_hbm, idx_hbm, out_hbm, idx_v, rows_v, sem):
        wid = lax.axis_index("s") * 2 + lax.axis_index("c")
        base = wid * per_w

        def body(i, carry):
            off = pl.multiple_of(base + i * CH, CH)
            pltpu.sync_copy(idx_hbm.at[pl.ds(off, CH)], idx_v)
            pltpu.async_copy(table_hbm.at[idx_v], rows_v, sem).wait()
            pltpu.sync_copy(rows_v, out_hbm.at[pl.ds(off, CH)])
            return carry

        lax.fori_loop(0, n_chunks, body, 0)

    return k(table, idx)


def _flat_idx(idx, B, M, K, N, TM):
    T = M // TM
    x = idx + (jnp.arange(B, dtype=jnp.int32) * N)[:, None, None]
    x = x.reshape(B, T, TM, K).transpose(0, 1, 3, 2)
    return x.reshape(B * M * K)


def _sa_tail_body(g2_ref, cx_ref, cf_ref, Wr_ref, c0_ref, g0_ref, be0_ref,
                  W1_ref, b1_ref, g1_ref, be1_ref,
                  Wres_ref, bres_ref, gres_ref, beres_ref,
                  Wfn_ref, Wrn_ref,
                  feat_ref, un_ref, *, K, TM, H, inv_r):
    cx = cx_ref[0]
    V = c0_ref[...] - _mm(cx, Wr_ref[...])
    Vt = jnp.concatenate([V] * K, axis=0)
    h = _gelu(_ln(g2_ref[...][:, :H] + Vt, g0_ref[...], be0_ref[...]))
    h1 = _ln(_mm(h, W1_ref[...]) + b1_ref[...], g1_ref[...], be1_ref[...])
    mx = h1[0:TM, :]
    for k in range(1, K):
        mx = jnp.maximum(mx, h1[k * TM:(k + 1) * TM, :])
    res = _ln(_mm(cf_ref[0], Wres_ref[...]) + bres_ref[...],
              gres_ref[...], beres_ref[...])
    out = _gelu(mx + res)
    feat_ref[0] = out
    un_ref[0] = _mm(out, Wfn_ref[...]) + _mm(cx, Wrn_ref[...])


def _sa_tail(G2, cxyz, cfeat, w, wnext, M, K, H, C2, inv_r, TM):
    B = cxyz.shape[0]
    T = M // TM
    Hg = G2.shape[1]
    Hn = wnext['Wf'].shape[1]
    body = functools.partial(_sa_tail_body, K=K, TM=TM, H=H, inv_r=inv_r)
    full = lambda a: pl.BlockSpec(a.shape, lambda bb, t: tuple(0 for _ in a.shape))
    return pl.pallas_call(
        body,
        grid=(B, T),
        in_specs=[
            pl.BlockSpec((TM * K, Hg), lambda bb, t: (bb * T + t, 0)),
            pl.BlockSpec((1, TM, 3), lambda bb, t: (bb, t, 0)),
            pl.BlockSpec((1, TM, cfeat.shape[-1]), lambda bb, t: (bb, t, 0)),
            full(w['Wr']), full(w['c0']), full(w['g0']), full(w['be0']),
            full(w['W1']), full(w['b1']), full(w['g1']), full(w['be1']),
            full(w['Wres']), full(w['bres']), full(w['gres']), full(w['beres']),
            full(wnext['Wf']), full(wnext['Wr']),
        ],
        out_specs=[
            pl.BlockSpec((1, TM, C2), lambda bb, t: (bb, t, 0)),
            pl.BlockSpec((1, TM, Hn), lambda bb, t: (bb, t, 0)),
        ],
        out_shape=[
            jax.ShapeDtypeStruct((B, M, C2), F32),
            jax.ShapeDtypeStruct((B, M, Hn), F32),
        ],
    )(G2, cxyz, cfeat, w['Wr'], w['c0'], w['g0'], w['be0'],
      w['W1'], w['b1'], w['g1'], w['be1'],
      w['Wres'], w['bres'], w['gres'], w['beres'],
      wnext['Wf'], wnext['Wr'])


def _irb_tail_body(g2_ref, cx_ref, f_ref, Wr_ref, c0_ref, g0_ref, be0_ref,
                   W1_ref, b1_ref, g1_ref, be1_ref,
                   Wc0_ref, bc0_ref, gc0_ref, bec0_ref,
                   Wc1_ref, bc1_ref, gc1_ref, bec1_ref,
                   Wfn_ref, Wrn_ref,
                   feat_ref, un_ref, *, K, TM, inv_r):
    cx = cx_ref[0]
    V = c0_ref[...] - _mm(cx, Wr_ref[...])
    Vt = jnp.concatenate([V] * K, axis=0)
    h = _gelu(_ln(g2_ref[...] + Vt, g0_ref[...], be0_ref[...]))
    h1 = _ln(_mm(h, W1_ref[...]) + b1_ref[...], g1_ref[...], be1_ref[...])
    mx = h1[0:TM, :]
    for k in range(1, K):
        mx = jnp.maximum(mx, h1[k * TM:(k + 1) * TM, :])
    hc = _gelu(_ln(_mm(mx, Wc0_ref[...]) + bc0_ref[...],
                   gc0_ref[...], bec0_ref[...]))
    hc = _ln(_mm(hc, Wc1_ref[...]) + bc1_ref[...], gc1_ref[...], bec1_ref[...])
    out = _gelu(hc + f_ref[0])
    feat_ref[0] = out
    un_ref[0] = _mm(out, Wfn_ref[...]) + _mm(cx, Wrn_ref[...])


def _final_tail_body(g2_ref, cx_ref, f_ref, Wr_ref, c0_ref, g0_ref, be0_ref,
                     W1_ref, b1_ref, g1_ref, be1_ref,
                     Wc0_ref, bc0_ref, gc0_ref, bec0_ref,
                     Wc1_ref, bc1_ref, gc1_ref, bec1_ref,
                     freqs_ref, gpW_ref, gpb_ref, gpg_ref, gpbe_ref,
                     opW_ref, opb_ref, opg_ref, opbe_ref,
                     out_ref, *, K, TM, inv_r):
    cx = cx_ref[0]
    V = c0_ref[...] - _mm(cx, Wr_ref[...])
    Vt = jnp.concatenate([V] * K, axis=0)
    h = _gelu(_ln(g2_ref[...] + Vt, g0_ref[...], be0_ref[...]))
    h1 = _ln(_mm(h, W1_ref[...]) + b1_ref[...], g1_ref[...], be1_ref[...])
    mx = h1[0:TM, :]
    for k in range(1, K):
        mx = jnp.maximum(mx, h1[k * TM:(k + 1) * TM, :])
    hc = _gelu(_ln(_mm(mx, Wc0_ref[...]) + bc0_ref[...],
                   gc0_ref[...], bec0_ref[...]))
    hc = _ln(_mm(hc, Wc1_ref[...]) + bc1_ref[...], gc1_ref[...], bec1_ref[...])
    feat = _gelu(hc + f_ref[0])
    fr = freqs_ref[...]
    parts = []
    for c in range(3):
        ang = cx[:, c:c + 1] * fr
        parts.append(jnp.sin(ang))
        parts.append(jnp.cos(ang))
    pe = jnp.concatenate(parts, axis=1)
    g = _gelu(_ln(_mm(pe, gpW_ref[...]) + gpb_ref[...],
                  gpg_ref[...], gpbe_ref[...]))
    feat = feat + g
    out_ref[0] = _ln(_mm(feat, opW_ref[...]) + opb_ref[...],
                     opg_ref[...], opbe_ref[...])


def _irb_tail(G2, cxyz, feat, w, wnext, M, K, H, C2, inv_r, TM, final=None):
    B = cxyz.shape[0]
    T = M // TM
    full = lambda a: pl.BlockSpec(a.shape, lambda bb, t: tuple(0 for _ in a.shape))
    common_in = [
        pl.BlockSpec((TM * K, H), lambda bb, t: (bb * T + t, 0)),
        pl.BlockSpec((1, TM, 3), lambda bb, t: (bb, t, 0)),
        pl.BlockSpec((1, TM, C2), lambda bb, t: (bb, t, 0)),
        full(w['Wr']), full(w['c0']), full(w['g0']), full(w['be0']),
        full(w['W1']), full(w['b1']), full(w['g1']), full(w['be1']),
        full(w['Wc0']), full(w['bc0']), full(w['gc0']), full(w['bec0']),
        full(w['Wc1']), full(w['bc1']), full(w['gc1']), full(w['bec1']),
    ]
    common_args = (G2, cxyz, feat, w['Wr'], w['c0'], w['g0'], w['be0'],
                   w['W1'], w['b1'], w['g1'], w['be1'],
                   w['Wc0'], w['bc0'], w['gc0'], w['bec0'],
                   w['Wc1'], w['bc1'], w['gc1'], w['bec1'])
    if final is None:
        Hn = wnext['Wf'].shape[1]
        body = functools.partial(_irb_tail_body, K=K, TM=TM, inv_r=inv_r)
        return pl.pallas_call(
            body,
            grid=(B, T),
            in_specs=common_in + [full(wnext['Wf']), full(wnext['Wr'])],
            out_specs=[
                pl.BlockSpec((1, TM, C2), lambda bb, t: (bb, t, 0)),
                pl.BlockSpec((1, TM, Hn), lambda bb, t: (bb, t, 0)),
            ],
            out_shape=[
                jax.ShapeDtypeStruct((B, M, C2), F32),
                jax.ShapeDtypeStruct((B, M, Hn), F32),
            ],
        )(*common_args, wnext['Wf'], wnext['Wr'])
    body = functools.partial(_final_tail_body, K=K, TM=TM, inv_r=inv_r)
    return pl.pallas_call(
        body,
        grid=(B, T),
        in_specs=common_in + [full(final[n]) for n in
                              ('freqs', 'gpW', 'gpb', 'gpg', 'gpbe',
                               'opW', 'opb', 'opg', 'opbe')],
        out_specs=pl.BlockSpec((1, TM, 256), lambda bb, t: (bb, t, 0)),
        out_shape=jax.ShapeDtypeStruct((B, M, 256), F32),
    )(*common_args, *(final[n] for n in
                      ('freqs', 'gpW', 'gpb', 'gpg', 'gpbe',
                       'opW', 'opb', 'opg', 'opbe')))


def _row(v):
    return v.reshape(1, -1)


def _prep_grouped(p, w0_key, cin_feat, radius):
    r = max(radius, 1e-6)
    W0 = p[w0_key]['W']
    Wrel, Wfeat, Wpos = W0[:3], W0[3:3 + cin_feat], W0[3 + cin_feat:]
    Wr = Wrel + p['rpe_W'] @ Wpos
    c0 = p['rpe_b'] @ Wpos + p[w0_key]['b']
    return {'Wf': Wfeat, 'Wr': Wr / r, 'c0': _row(c0),
            'g0': _row(p[w0_key]['g']), 'be0': _row(p[w0_key]['be'])}


def kernel(pointcloud, params):
    B, N0, _ = pointcloud.shape
    N1, N2 = N0 // 2, N0 // 4
    K1, K2 = 24, 32
    R1, R2 = 0.08, 0.16

    xyz = pointcloud[..., :3]
    pxt = jnp.transpose(xyz, (0, 2, 1))

    p1 = params['s1_sa']
    pi1 = params['s1_irb0']
    p2 = params['s2_sa']
    pi2 = params['s2_irb0']

    w1 = _prep_grouped(p1, 'mlp0', 64, R1)
    wi1 = _prep_grouped(pi1, 'la0', 128, R1)
    w2 = _prep_grouped(p2, 'mlp0', 128, R2)
    wi2 = _prep_grouped(pi2, 'la0', 256, R2)

    def mlp_w(q, name):
        return {name + 'W': q['W'], name + 'b': _row(q['b']),
                name + 'g': _row(q['g']), name + 'be': _row(q['be'])}

    sa1_w = dict(w1, W1=p1['mlp1']['W'], b1=_row(p1['mlp1']['b']),
                 g1=_row(p1['mlp1']['g']), be1=_row(p1['mlp1']['be']),
                 Wres=p1['res']['W'], bres=_row(p1['res']['b']),
                 gres=_row(p1['res']['g']), beres=_row(p1['res']['be']))
    irb1_w = dict(wi1, W1=pi1['la1']['W'], b1=_row(pi1['la1']['b']),
                  g1=_row(pi1['la1']['g']), be1=_row(pi1['la1']['be']),
                  Wc0=pi1['cm0']['W'], bc0=_row(pi1['cm0']['b']),
                  gc0=_row(pi1['cm0']['g']), bec0=_row(pi1['cm0']['be']),
                  Wc1=pi1['cm1']['W'], bc1=_row(pi1['cm1']['b']),
                  gc1=_row(pi1['cm1']['g']), bec1=_row(pi1['cm1']['be']))
    sa2_w = dict(w2, W1=p2['mlp1']['W'], b1=_row(p2['mlp1']['b']),
                 g1=_row(p2['mlp1']['g']), be1=_row(p2['mlp1']['be']),
                 Wres=p2['res']['W'], bres=_row(p2['res']['b']),
                 gres=_row(p2['res']['g']), beres=_row(p2['res']['be']))
    irb2_w = dict(wi2, W1=pi2['la1']['W'], b1=_row(pi2['la1']['b']),
                  g1=_row(pi2['la1']['g']), be1=_row(pi2['la1']['be']),
                  Wc0=pi2['cm0']['W'], bc0=_row(pi2['cm0']['b']),
                  gc0=_row(pi2['cm0']['g']), bec0=_row(pi2['cm0']['be']),
                  Wc1=pi2['cm1']['W'], bc1=_row(pi2['cm1']['b']),
                  gc1=_row(pi2['cm1']['g']), bec1=_row(pi2['cm1']['be']))

    half = 16
    freqs = jnp.exp(-jnp.log(10000.0) *
                    jnp.arange(half, dtype=F32) / (half - 1)).reshape(1, half)
    final_w = {'freqs': freqs,
               'gpW': params['gp_W'], 'gpb': _row(params['gp_b']),
               'gpg': _row(params['gp_g']), 'gpbe': _row(params['gp_be']),
               'opW': params['op_W'], 'opb': _row(params['op_b']),
               'opg': _row(params['op_g']), 'opbe': _row(params['op_be'])}

    xp = xyz[..., 0].reshape(-1)
    yp = xyz[..., 1].reshape(-1)
    zp = xyz[..., 2].reshape(-1)
    idx1 = _ball_query(xyz, pxt, xp, yp, zp, N1, N0, K1, R1)
    idx2 = _ball_query(xyz, pxt, xp, yp, zp, N1, N1, K1, R1)
    idx3 = _ball_query(xyz, pxt, xp, yp, zp, N2, N1, K2, R2)
    idx4 = _ball_query(xyz, pxt, xp, yp, zp, N2, N2, K2, R2)

    TM1, TMi1, TM2, TMi2 = 256, 128, 128, 64
    fidx1 = _flat_idx(idx1, B, N1, K1, N0, TM1)
    fidx2 = _flat_idx(idx2, B, N1, K1, N1, TMi1)
    fidx3 = _flat_idx(idx3, B, N2, K2, N1, TM2)
    fidx4 = _flat_idx(idx4, B, N2, K2, N2, TMi2)

    pad128 = lambda a: jnp.pad(a, ((0, 0), (0, 128 - a.shape[1])))
    feat0, U1 = _head(pointcloud, params['s0_head']['W'],
                      _row(params['s0_head']['b']), _row(params['s0_head']['g']),
                      _row(params['s0_head']['be']),
                      pad128(w1['Wf']), pad128(w1['Wr']))

    G1 = _sc_gather(U1.reshape(B * N0, 128), fidx1)
    feat1, U2 = _sa_tail(G1, xyz, feat0, sa1_w, wi1, N1, K1, 64, 128,
                         1.0 / max(R1, 1e-6), TM1)

    G2 = _sc_gather(U2.reshape(B * N1, 256), fidx2)
    feat1b, U3 = _irb_tail(G2, xyz, feat1, irb1_w, w2, N1, K1, 256, 128,
                           1.0 / max(R1, 1e-6), TMi1)

    G3 = _sc_gather(U3.reshape(B * N1, 128), fidx3)
    feat2, U4 = _sa_tail(G3, xyz, feat1b, sa2_w, wi2, N2, K2, 128, 256,
                         1.0 / max(R2, 1e-6), TM2)

    G4 = _sc_gather(U4.reshape(B * N2, 512), fidx4)
    out = _irb_tail(G4, xyz, feat2, irb2_w, None, N2, K2, 512, 256,
                    1.0 / max(R2, 1e-6), TMi2, final=final_w)
    return out


# speedup vs baseline: 1.0206x; 1.0206x over previous
import functools
import math

import jax
import jax.numpy as jnp
from jax import lax
from jax.experimental import pallas as pl
from jax.experimental.pallas import tpu as pltpu
from jax.experimental.pallas import tpu_sc as plsc

F32 = jnp.float32
_EPS = 1e-5
_INV_SQRT2 = 0.7071067811865476


def _gelu(x):
    return 0.5 * x * (1.0 + lax.erf(x * _INV_SQRT2))


def _ln(x, g, b):
    m = jnp.mean(x, -1, keepdims=True)
    xc = x - m
    v = jnp.mean(xc * xc, -1, keepdims=True)
    return xc / jnp.sqrt(v + _EPS) * g + b


def _mm(a, b):
    return lax.dot_general(a, b, (((1,), (0,)), ((), ())),
                           preferred_element_type=F32)


def _head_body(pc_ref, W_ref, b_ref, g_ref, be_ref, Wf_ref, Wr_ref,
               feat_ref, u_ref):
    pc = pc_ref[0]
    f = _gelu(_ln(_mm(pc, W_ref[...]) + b_ref[...], g_ref[...], be_ref[...]))
    feat_ref[0] = f
    xyz = pc[:, :3]
    u_ref[0] = _mm(f, Wf_ref[...]) + _mm(xyz, Wr_ref[...])


def _head(pc, W, b, g, be, Wf, Wr_scaled, TN=1024):
    B, N, _ = pc.shape
    H = Wf.shape[1]
    grid = (B, N // TN)
    return pl.pallas_call(
        _head_body,
        grid=grid,
        in_specs=[
            pl.BlockSpec((1, TN, 6), lambda bb, t: (bb, t, 0)),
            pl.BlockSpec(W.shape, lambda bb, t: (0, 0)),
            pl.BlockSpec(b.shape, lambda bb, t: (0, 0)),
            pl.BlockSpec(g.shape, lambda bb, t: (0, 0)),
            pl.BlockSpec(be.shape, lambda bb, t: (0, 0)),
            pl.BlockSpec(Wf.shape, lambda bb, t: (0, 0)),
            pl.BlockSpec(Wr_scaled.shape, lambda bb, t: (0, 0)),
        ],
        out_specs=[
            pl.BlockSpec((1, TN, 64), lambda bb, t: (bb, t, 0)),
            pl.BlockSpec((1, TN, H), lambda bb, t: (bb, t, 0)),
        ],
        out_shape=[
            jax.ShapeDtypeStruct((B, N, 64), F32),
            jax.ShapeDtypeStruct((B, N, H), F32),
        ],
    )(pc, W, b, g, be, Wf, Wr_scaled)


_CH = 16


def _dc_body(pts_ref, cxt_ref, out_ref, *, N, TML):
    P = pts_ref[0]
    C = cxt_ref[0]
    pn2 = jnp.sum(P * P, axis=1, keepdims=True)
    cn2 = jnp.sum(C * C, axis=0, keepdims=True)
    dot = lax.dot_general(P, C, (((1,), (0,)), ((), ())),
                          preferred_element_type=F32,
                          precision=lax.Precision.HIGHEST)
    d = pn2 + cn2 - 2.0 * dot
    dc = jnp.min(d.reshape(N // _CH, _CH, TML), axis=1)
    out_ref[0] = dc


def _dc_chunkmin(xyz, pxt, M, N, TML=128):
    B = xyz.shape[0]
    NC = N // _CH
    body = functools.partial(_dc_body, N=N, TML=TML)
    return pl.pallas_call(
        body,
        grid=(B, M // TML),
        in_specs=[
            pl.BlockSpec((1, N, 3), lambda bb, t: (bb, 0, 0)),
            pl.BlockSpec((1, 3, TML), lambda bb, t: (bb, 0, t)),
        ],
        out_specs=pl.BlockSpec((1, NC, TML), lambda bb, t: (bb, 0, t)),
        out_shape=jax.ShapeDtypeStruct((B, NC, M), F32),
    )(xyz, pxt)


def _sc_ballquery(dc_flat, xp, yp, zp, B, M, N, NS, K, radius):
    NC = N // _CH
    r2 = radius * radius
    r2m = r2 * 1.0001 + 1e-5
    NW = 32
    RPW = (B * M) // NW
    assert (B * M) % NW == 0 and M % RPW == 0
    GMAX = NC // 16
    CAP = N + 16
    BIGF = jnp.float32(1e30)
    BIGI = jnp.int32(2 ** 30)

    mesh = plsc.VectorSubcoreMesh(core_axis_name="c", subcore_axis_name="s")

    @functools.partial(
        pl.kernel,
        mesh=mesh,
        out_type=jax.ShapeDtypeStruct((B * M * K,), jnp.int32),
        compiler_params=pltpu.CompilerParams(needs_layout_passes=False),
        scratch_types=[
            pltpu.VMEM((RPW * NC,), F32),
            pltpu.VMEM((N,), F32),
            pltpu.VMEM((N,), F32),
            pltpu.VMEM((N,), F32),
            pltpu.VMEM((2 * (NC + 16),), jnp.int32),
            pltpu.VMEM((2 * CAP,), jnp.int32),
            pltpu.VMEM((2 * CAP,), F32),
            pltpu.VMEM((RPW * K + 16,), jnp.int32),
            pltpu.SemaphoreType.DMA,
        ],
    )
    def kq(dc_hbm, xp_hbm, yp_hbm, zp_hbm, out_hbm,
           dcv, xv, yv, zv, ccv, piv, pdv, outv, sem):
        wid = lax.axis_index("s") * 2 + lax.axis_index("c")
        row0 = wid * RPW
        b = row0 // M
        cbase = row0 - b * M
        pltpu.sync_copy(dc_hbm.at[pl.ds(row0 * NC, RPW * NC)], dcv)
        pltpu.sync_copy(xp_hbm.at[pl.ds(b * NS, N)], xv)
        pltpu.sync_copy(yp_hbm.at[pl.ds(b * NS, N)], yv)
        pltpu.sync_copy(zp_hbm.at[pl.ds(b * NS, N)], zv)
        lanes = lax.broadcasted_iota(jnp.int32, (16,), 0)

        @plsc.parallel_loop(0, RPW, 1, unroll=2)
        def row_body(r):
            par = r % 2
            coff = par * (NC + 16)
            poff = par * CAP
            cloc = cbase + r
            csp = jnp.full((16,), cloc, jnp.int32)
            cx = plsc.load_gather(xv, [csp])
            cy = plsc.load_gather(yv, [csp])
            cz = plsc.load_gather(zv, [csp])

            def p1(c, carry):
                ncc_s, vm, im = carry
                v = dcv[pl.ds(r * NC + c * 16, 16)]
                ids = lanes + c * 16
                mask = v <= r2m
                cnt = plsc.all_reduce_population_count(mask)
                plsc.store_compressed(ccv.at[pl.ds(coff + ncc_s, 16)], ids,
                                      mask=mask)
                im = jnp.where(v < vm, ids, im)
                vm = jnp.minimum(vm, v)
                return ncc_s + cnt[0], vm, im

            ncc, vm, im = lax.fori_loop(
                0, GMAX, p1,
                (jnp.int32(0), jnp.full((16,), BIGF),
                 jnp.zeros((16,), jnp.int32)))
            mch = jnp.min(vm)
            amc = jnp.min(jnp.where(vm == mch, im, BIGI))
            plsc.store_scatter(ccv, [coff + lanes],
                               jnp.full((16,), amc, jnp.int32),
                               mask=(lanes == 0) & (ncc == 0))
            ncc = jnp.maximum(ncc, 1)

            def p2(g, carry):
                np_s, tmv, tav = carry
                gids = plsc.load_gather(ccv, [coff + lanes + g * 16])
                valid = (lanes + g * 16) < ncc
                gids = jnp.where(valid, gids, 0)
                for j in range(_CH):
                    pt = gids * _CH + j
                    dx = plsc.load_gather(xv, [pt]) - cx
                    dy = plsc.load_gather(yv, [pt]) - cy
                    dz = plsc.load_gather(zv, [pt]) - cz
                    d2 = dx * dx + dy * dy + dz * dz
                    d2 = jnp.where(valid, d2, BIGF)
                    inr = d2 <= r2
                    plsc.store_compressed(piv.at[pl.ds(poff + np_s, 16)], pt,
                                          mask=inr)
                    plsc.store_compressed(pdv.at[pl.ds(poff + np_s, 16)], d2,
                                          mask=inr)
                    np_s = np_s + plsc.all_reduce_population_count(inr)[0]
                    tav = jnp.where(d2 < tmv, pt, tav)
                    tmv = jnp.minimum(tmv, d2)
                return np_s, tmv, tav

            np_, tmv, tav = lax.fori_loop(
                0, (ncc + 15) // 16, p2,
                (jnp.int32(0), jnp.full((16,), BIGF),
                 jnp.zeros((16,), jnp.int32)))
            tmn = jnp.min(tmv)
            pad = jnp.min(jnp.where(tmv == tmn, tav, BIGI))
            padv = jnp.full((16,), pad, jnp.int32)

            off = r * K
            for s in range(0, K, 16):
                plsc.store_scatter(outv, [off + s + lanes], padv, mask=lanes < (K - s))

            @pl.when(np_ <= K)
            def _small():
                def emit(g, _c):
                    pos = g * 16 + lanes
                    vals = plsc.load_gather(piv, [poff + jnp.where(pos < np_, pos, 0)])
                    plsc.store_scatter(outv, [off + pos], vals, mask=pos < np_)
                    return _c
                lax.fori_loop(0, (np_ + 15) // 16, emit, 0)

            @pl.when(np_ > K)
            def _big():
                gsel = (np_ + 15) // 16

                def pick(kk, _c):
                    def scan(g, carry):
                        mv, pv = carry
                        pos = g * 16 + lanes
                        v = plsc.load_gather(pdv, [poff + jnp.where(pos < np_, pos, 0)])
                        v = jnp.where(pos < np_, v, BIGF)
                        pv = jnp.where(v < mv, pos, pv)
                        mv = jnp.minimum(mv, v)
                        return mv, pv
                    mv, pv = lax.fori_loop(
                        0, gsel, scan,
                        (jnp.full((16,), BIGF), jnp.zeros((16,), jnp.int32)))
                    mn = jnp.min(mv)
                    sel = jnp.min(jnp.where(mv == mn, pv, BIGI))
                    selv = jnp.full((16,), sel, jnp.int32)
                    val = plsc.load_gather(piv, [poff + selv])
                    plsc.store_scatter(outv, [off + kk + lanes], val, mask=lanes == 0)
                    plsc.store_scatter(pdv, [poff + selv], jnp.full((16,), BIGF),
                                       mask=lanes == 0)
                    return _c
                lax.fori_loop(0, K, pick, 0)


        pltpu.sync_copy(outv.at[pl.ds(0, RPW * K)],
                        out_hbm.at[pl.ds(row0 * K, RPW * K)])

    return kq(dc_flat, xp, yp, zp)


def _ball_query(xyz, pxt, xp, yp, zp, M, N, K, radius):
    B, NS = xyz.shape[0], xyz.shape[1]
    dct = _dc_chunkmin(xyz[:, :N], pxt[:, :, :M], M, N)
    dc_flat = jnp.transpose(dct, (0, 2, 1)).reshape(-1)
    idx = _sc_ballquery(dc_flat, xp, yp, zp, B, M, N, NS, K, radius)
    return idx.reshape(B, M, K)


def _sc_gather(table, idx):
    R = idx.shape[0]
    H = table.shape[1]
    NW = 32
    per_w = R // NW
    CH = 128
    n_chunks = per_w // CH
    assert per_w % CH == 0 and R % NW == 0

    mesh = plsc.VectorSubcoreMesh(core_axis_name="c", subcore_axis_name="s")

    @functools.partial(
        pl.kernel,
        mesh=mesh,
        out_type=jax.ShapeDtypeStruct((R, H), F32),
        scratch_types=[
            pltpu.VMEM((CH,), jnp.int32),
            pltpu.VMEM((CH, H), F32),
            pltpu.SemaphoreType.DMA,
        ],
    )
    def k(table_hbm, idx_hbm, out_hbm, idx_v, rows_v, sem):
        wid = lax.axis_index("s") * 2 + lax.axis_index("c")
        base = wid * per_w

        def body(i, carry):
            off = pl.multiple_of(base + i * CH, CH)
            pltpu.sync_copy(idx_hbm.at[pl.ds(off, CH)], idx_v)
            pltpu.async_copy(table_hbm.at[idx_v], rows_v, sem).wait()
            pltpu.sync_copy(rows_v, out_hbm.at[pl.ds(off, CH)])
            return carry

        lax.fori_loop(0, n_chunks, body, 0)

    return k(table, idx)


def _flat_idx(idx, B, M, K, N, TM):
    T = M // TM
    x = idx + (jnp.arange(B, dtype=jnp.int32) * N)[:, None, None]
    x = x.reshape(B, T, TM, K).transpose(0, 1, 3, 2)
    return x.reshape(B * M * K)


def _sa_tail_body(g2_ref, cx_ref, cf_ref, Wr_ref, c0_ref, g0_ref, be0_ref,
                  W1_ref, b1_ref, g1_ref, be1_ref,
                  Wres_ref, bres_ref, gres_ref, beres_ref,
                  Wfn_ref, Wrn_ref,
                  feat_ref, un_ref, *, K, TM, H, inv_r):
    cx = cx_ref[0]
    V = c0_ref[...] - _mm(cx, Wr_ref[...])
    Vt = jnp.concatenate([V] * K, axis=0)
    h = _gelu(_ln(g2_ref[...][:, :H] + Vt, g0_ref[...], be0_ref[...]))
    h1 = _ln(_mm(h, W1_ref[...]) + b1_ref[...], g1_ref[...], be1_ref[...])
    mx = h1[0:TM, :]
    for k in range(1, K):
        mx = jnp.maximum(mx, h1[k * TM:(k + 1) * TM, :])
    res = _ln(_mm(cf_ref[0], Wres_ref[...]) + bres_ref[...],
              gres_ref[...], beres_ref[...])
    out = _gelu(mx + res)
    feat_ref[0] = out
    un_ref[0] = _mm(out, Wfn_ref[...]) + _mm(cx, Wrn_ref[...])


def _sa_tail(G2, cxyz, cfeat, w, wnext, M, K, H, C2, inv_r, TM):
    B = cxyz.shape[0]
    T = M // TM
    Hg = G2.shape[1]
    Hn = wnext['Wf'].shape[1]
    body = functools.partial(_sa_tail_body, K=K, TM=TM, H=H, inv_r=inv_r)
    full = lambda a: pl.BlockSpec(a.shape, lambda bb, t: tuple(0 for _ in a.shape))
    return pl.pallas_call(
        body,
        grid=(B, T),
        in_specs=[
            pl.BlockSpec((TM * K, Hg), lambda bb, t: (bb * T + t, 0)),
            pl.BlockSpec((1, TM, 3), lambda bb, t: (bb, t, 0)),
            pl.BlockSpec((1, TM, cfeat.shape[-1]), lambda bb, t: (bb, t, 0)),
            full(w['Wr']), full(w['c0']), full(w['g0']), full(w['be0']),
            full(w['W1']), full(w['b1']), full(w['g1']), full(w['be1']),
            full(w['Wres']), full(w['bres']), full(w['gres']), full(w['beres']),
            full(wnext['Wf']), full(wnext['Wr']),
        ],
        out_specs=[
            pl.BlockSpec((1, TM, C2), lambda bb, t: (bb, t, 0)),
            pl.BlockSpec((1, TM, Hn), lambda bb, t: (bb, t, 0)),
        ],
        out_shape=[
            jax.ShapeDtypeStruct((B, M, C2), F32),
            jax.ShapeDtypeStruct((B, M, Hn), F32),
        ],
    )(G2, cxyz, cfeat, w['Wr'], w['c0'], w['g0'], w['be0'],
      w['W1'], w['b1'], w['g1'], w['be1'],
      w['Wres'], w['bres'], w['gres'], w['beres'],
      wnext['Wf'], wnext['Wr'])


def _irb_tail_body(g2_ref, cx_ref, f_ref, Wr_ref, c0_ref, g0_ref, be0_ref,
                   W1_ref, b1_ref, g1_ref, be1_ref,
                   Wc0_ref, bc0_ref, gc0_ref, bec0_ref,
                   Wc1_ref, bc1_ref, gc1_ref, bec1_ref,
                   Wfn_ref, Wrn_ref,
                   feat_ref, un_ref, *, K, TM, inv_r):
    cx = cx_ref[0]
    V = c0_ref[...] - _mm(cx, Wr_ref[...])
    Vt = jnp.concatenate([V] * K, axis=0)
    h = _gelu(_ln(g2_ref[...] + Vt, g0_ref[...], be0_ref[...]))
    h1 = _ln(_mm(h, W1_ref[...]) + b1_ref[...], g1_ref[...], be1_ref[...])
    mx = h1[0:TM, :]
    for k in range(1, K):
        mx = jnp.maximum(mx, h1[k * TM:(k + 1) * TM, :])
    hc = _gelu(_ln(_mm(mx, Wc0_ref[...]) + bc0_ref[...],
                   gc0_ref[...], bec0_ref[...]))
    hc = _ln(_mm(hc, Wc1_ref[...]) + bc1_ref[...], gc1_ref[...], bec1_ref[...])
    out = _gelu(hc + f_ref[0])
    feat_ref[0] = out
    un_ref[0] = _mm(out, Wfn_ref[...]) + _mm(cx, Wrn_ref[...])


def _final_tail_body(g2_ref, cx_ref, f_ref, Wr_ref, c0_ref, g0_ref, be0_ref,
                     W1_ref, b1_ref, g1_ref, be1_ref,
                     Wc0_ref, bc0_ref, gc0_ref, bec0_ref,
                     Wc1_ref, bc1_ref, gc1_ref, bec1_ref,
                     freqs_ref, gpW_ref, gpb_ref, gpg_ref, gpbe_ref,
                     opW_ref, opb_ref, opg_ref, opbe_ref,
                     out_ref, *, K, TM, inv_r):
    cx = cx_ref[0]
    V = c0_ref[...] - _mm(cx, Wr_ref[...])
    Vt = jnp.concatenate([V] * K, axis=0)
    h = _gelu(_ln(g2_ref[...] + Vt, g0_ref[...], be0_ref[...]))
    h1 = _ln(_mm(h, W1_ref[...]) + b1_ref[...], g1_ref[...], be1_ref[...])
    mx = h1[0:TM, :]
    for k in range(1, K):
        mx = jnp.maximum(mx, h1[k * TM:(k + 1) * TM, :])
    hc = _gelu(_ln(_mm(mx, Wc0_ref[...]) + bc0_ref[...],
                   gc0_ref[...], bec0_ref[...]))
    hc = _ln(_mm(hc, Wc1_ref[...]) + bc1_ref[...], gc1_ref[...], bec1_ref[...])
    feat = _gelu(hc + f_ref[0])
    fr = freqs_ref[...]
    parts = []
    for c in range(3):
        ang = cx[:, c:c + 1] * fr
        parts.append(jnp.sin(ang))
        parts.append(jnp.cos(ang))
    pe = jnp.concatenate(parts, axis=1)
    g = _gelu(_ln(_mm(pe, gpW_ref[...]) + gpb_ref[...],
                  gpg_ref[...], gpbe_ref[...]))
    feat = feat + g
    out_ref[0] = _ln(_mm(feat, opW_ref[...]) + opb_ref[...],
                     opg_ref[...], opbe_ref[...])


def _irb_tail(G2, cxyz, feat, w, wnext, M, K, H, C2, inv_r, TM, final=None):
    B = cxyz.shape[0]
    T = M // TM
    full = lambda a: pl.BlockSpec(a.shape, lambda bb, t: tuple(0 for _ in a.shape))
    common_in = [
        pl.BlockSpec((TM * K, H), lambda bb, t: (bb * T + t, 0)),
        pl.BlockSpec((1, TM, 3), lambda bb, t: (bb, t, 0)),
        pl.BlockSpec((1, TM, C2), lambda bb, t: (bb, t, 0)),
        full(w['Wr']), full(w['c0']), full(w['g0']), full(w['be0']),
        full(w['W1']), full(w['b1']), full(w['g1']), full(w['be1']),
        full(w['Wc0']), full(w['bc0']), full(w['gc0']), full(w['bec0']),
        full(w['Wc1']), full(w['bc1']), full(w['gc1']), full(w['bec1']),
    ]
    common_args = (G2, cxyz, feat, w['Wr'], w['c0'], w['g0'], w['be0'],
                   w['W1'], w['b1'], w['g1'], w['be1'],
                   w['Wc0'], w['bc0'], w['gc0'], w['bec0'],
                   w['Wc1'], w['bc1'], w['gc1'], w['bec1'])
    if final is None:
        Hn = wnext['Wf'].shape[1]
        body = functools.partial(_irb_tail_body, K=K, TM=TM, inv_r=inv_r)
        return pl.pallas_call(
            body,
            grid=(B, T),
            in_specs=common_in + [full(wnext['Wf']), full(wnext['Wr'])],
            out_specs=[
                pl.BlockSpec((1, TM, C2), lambda bb, t: (bb, t, 0)),
                pl.BlockSpec((1, TM, Hn), lambda bb, t: (bb, t, 0)),
            ],
            out_shape=[
                jax.ShapeDtypeStruct((B, M, C2), F32),
                jax.ShapeDtypeStruct((B, M, Hn), F32),
            ],
        )(*common_args, wnext['Wf'], wnext['Wr'])
    body = functools.partial(_final_tail_body, K=K, TM=TM, inv_r=inv_r)
    return pl.pallas_call(
        body,
        grid=(B, T),
        in_specs=common_in + [full(final[n]) for n in
                              ('freqs', 'gpW', 'gpb', 'gpg', 'gpbe',
                               'opW', 'opb', 'opg', 'opbe')],
        out_specs=pl.BlockSpec((1, TM, 256), lambda bb, t: (bb, t, 0)),
        out_shape=jax.ShapeDtypeStruct((B, M, 256), F32),
    )(*common_args, *(final[n] for n in
                      ('freqs', 'gpW', 'gpb', 'gpg', 'gpbe',
                       'opW', 'opb', 'opg', 'opbe')))


def _row(v):
    return v.reshape(1, -1)


def _prep_grouped(p, w0_key, cin_feat, radius):
    r = max(radius, 1e-6)
    W0 = p[w0_key]['W']
    Wrel, Wfeat, Wpos = W0[:3], W0[3:3 + cin_feat], W0[3 + cin_feat:]
    Wr = Wrel + p['rpe_W'] @ Wpos
    c0 = p['rpe_b'] @ Wpos + p[w0_key]['b']
    return {'Wf': Wfeat, 'Wr': Wr / r, 'c0': _row(c0),
            'g0': _row(p[w0_key]['g']), 'be0': _row(p[w0_key]['be'])}


def kernel(pointcloud, params):
    B, N0, _ = pointcloud.shape
    N1, N2 = N0 // 2, N0 // 4
    K1, K2 = 24, 32
    R1, R2 = 0.08, 0.16

    xyz = pointcloud[..., :3]
    pxt = jnp.transpose(xyz, (0, 2, 1))

    p1 = params['s1_sa']
    pi1 = params['s1_irb0']
    p2 = params['s2_sa']
    pi2 = params['s2_irb0']

    w1 = _prep_grouped(p1, 'mlp0', 64, R1)
    wi1 = _prep_grouped(pi1, 'la0', 128, R1)
    w2 = _prep_grouped(p2, 'mlp0', 128, R2)
    wi2 = _prep_grouped(pi2, 'la0', 256, R2)

    def mlp_w(q, name):
        return {name + 'W': q['W'], name + 'b': _row(q['b']),
                name + 'g': _row(q['g']), name + 'be': _row(q['be'])}

    sa1_w = dict(w1, W1=p1['mlp1']['W'], b1=_row(p1['mlp1']['b']),
                 g1=_row(p1['mlp1']['g']), be1=_row(p1['mlp1']['be']),
                 Wres=p1['res']['W'], bres=_row(p1['res']['b']),
                 gres=_row(p1['res']['g']), beres=_row(p1['res']['be']))
    irb1_w = dict(wi1, W1=pi1['la1']['W'], b1=_row(pi1['la1']['b']),
                  g1=_row(pi1['la1']['g']), be1=_row(pi1['la1']['be']),
                  Wc0=pi1['cm0']['W'], bc0=_row(pi1['cm0']['b']),
                  gc0=_row(pi1['cm0']['g']), bec0=_row(pi1['cm0']['be']),
                  Wc1=pi1['cm1']['W'], bc1=_row(pi1['cm1']['b']),
                  gc1=_row(pi1['cm1']['g']), bec1=_row(pi1['cm1']['be']))
    sa2_w = dict(w2, W1=p2['mlp1']['W'], b1=_row(p2['mlp1']['b']),
                 g1=_row(p2['mlp1']['g']), be1=_row(p2['mlp1']['be']),
                 Wres=p2['res']['W'], bres=_row(p2['res']['b']),
                 gres=_row(p2['res']['g']), beres=_row(p2['res']['be']))
    irb2_w = dict(wi2, W1=pi2['la1']['W'], b1=_row(pi2['la1']['b']),
                  g1=_row(pi2['la1']['g']), be1=_row(pi2['la1']['be']),
                  Wc0=pi2['cm0']['W'], bc0=_row(pi2['cm0']['b']),
                  gc0=_row(pi2['cm0']['g']), bec0=_row(pi2['cm0']['be']),
                  Wc1=pi2['cm1']['W'], bc1=_row(pi2['cm1']['b']),
                  gc1=_row(pi2['cm1']['g']), bec1=_row(pi2['cm1']['be']))

    half = 16
    freqs = jnp.exp(-jnp.log(10000.0) *
                    jnp.arange(half, dtype=F32) / (half - 1)).reshape(1, half)
    final_w = {'freqs': freqs,
               'gpW': params['gp_W'], 'gpb': _row(params['gp_b']),
               'gpg': _row(params['gp_g']), 'gpbe': _row(params['gp_be']),
               'opW': params['op_W'], 'opb': _row(params['op_b']),
               'opg': _row(params['op_g']), 'opbe': _row(params['op_be'])}

    xp = xyz[..., 0].reshape(-1)
    yp = xyz[..., 1].reshape(-1)
    zp = xyz[..., 2].reshape(-1)
    idx1 = _ball_query(xyz, pxt, xp, yp, zp, N1, N0, K1, R1)
    idx2 = _ball_query(xyz, pxt, xp, yp, zp, N1, N1, K1, R1)
    idx3 = _ball_query(xyz, pxt, xp, yp, zp, N2, N1, K2, R2)
    idx4 = _ball_query(xyz, pxt, xp, yp, zp, N2, N2, K2, R2)

    TM1, TMi1, TM2, TMi2 = 256, 128, 128, 64
    fidx1 = _flat_idx(idx1, B, N1, K1, N0, TM1)
    fidx2 = _flat_idx(idx2, B, N1, K1, N1, TMi1)
    fidx3 = _flat_idx(idx3, B, N2, K2, N1, TM2)
    fidx4 = _flat_idx(idx4, B, N2, K2, N2, TMi2)

    pad128 = lambda a: jnp.pad(a, ((0, 0), (0, 128 - a.shape[1])))
    feat0, U1 = _head(pointcloud, params['s0_head']['W'],
                      _row(params['s0_head']['b']), _row(params['s0_head']['g']),
                      _row(params['s0_head']['be']),
                      pad128(w1['Wf']), pad128(w1['Wr']))

    G1 = _sc_gather(U1.reshape(B * N0, 128), fidx1)
    feat1, U2 = _sa_tail(G1, xyz, feat0, sa1_w, wi1, N1, K1, 64, 128,
                         1.0 / max(R1, 1e-6), TM1)

    G2 = _sc_gather(U2.reshape(B * N1, 256), fidx2)
    feat1b, U3 = _irb_tail(G2, xyz, feat1, irb1_w, w2, N1, K1, 256, 128,
                           1.0 / max(R1, 1e-6), TMi1)

    G3 = _sc_gather(U3.reshape(B * N1, 128), fidx3)
    feat2, U4 = _sa_tail(G3, xyz, feat1b, sa2_w, wi2, N2, K2, 128, 256,
                         1.0 / max(R2, 1e-6), TM2)

    G4 = _sc_gather(U4.reshape(B * N2, 512), fidx4)
    out = _irb_tail(G4, xyz, feat2, irb2_w, None, N2, K2, 512, 256,
                    1.0 / max(R2, 1e-6), TMi2, final=final_w)
    return out


# speedup vs baseline: 1.0645x; 1.0430x over previous
import functools
import math

import jax
import jax.numpy as jnp
from jax import lax
from jax.experimental import pallas as pl
from jax.experimental.pallas import tpu as pltpu
from jax.experimental.pallas import tpu_sc as plsc

F32 = jnp.float32
_EPS = 1e-5
_INV_SQRT2 = 0.7071067811865476


def _gelu(x):
    return 0.5 * x * (1.0 + lax.erf(x * _INV_SQRT2))


def _ln(x, g, b):
    m = jnp.mean(x, -1, keepdims=True)
    xc = x - m
    v = jnp.mean(xc * xc, -1, keepdims=True)
    return xc / jnp.sqrt(v + _EPS) * g + b


def _mm(a, b):
    return lax.dot_general(a, b, (((1,), (0,)), ((), ())),
                           preferred_element_type=F32)


def _head_body(pc_ref, W_ref, b_ref, g_ref, be_ref, Wf_ref, Wr_ref,
               feat_ref, u_ref):
    pc = pc_ref[0]
    f = _gelu(_ln(_mm(pc, W_ref[...]) + b_ref[...], g_ref[...], be_ref[...]))
    feat_ref[0] = f
    xyz = pc[:, :3]
    u_ref[0] = _mm(f, Wf_ref[...]) + _mm(xyz, Wr_ref[...])


def _head(pc, W, b, g, be, Wf, Wr_scaled, TN=1024):
    B, N, _ = pc.shape
    H = Wf.shape[1]
    grid = (B, N // TN)
    return pl.pallas_call(
        _head_body,
        grid=grid,
        in_specs=[
            pl.BlockSpec((1, TN, 6), lambda bb, t: (bb, t, 0)),
            pl.BlockSpec(W.shape, lambda bb, t: (0, 0)),
            pl.BlockSpec(b.shape, lambda bb, t: (0, 0)),
            pl.BlockSpec(g.shape, lambda bb, t: (0, 0)),
            pl.BlockSpec(be.shape, lambda bb, t: (0, 0)),
            pl.BlockSpec(Wf.shape, lambda bb, t: (0, 0)),
            pl.BlockSpec(Wr_scaled.shape, lambda bb, t: (0, 0)),
        ],
        out_specs=[
            pl.BlockSpec((1, TN, 64), lambda bb, t: (bb, t, 0)),
            pl.BlockSpec((1, TN, H), lambda bb, t: (bb, t, 0)),
        ],
        out_shape=[
            jax.ShapeDtypeStruct((B, N, 64), F32),
            jax.ShapeDtypeStruct((B, N, H), F32),
        ],
    )(pc, W, b, g, be, Wf, Wr_scaled)


_CH = 16


def _dc_body(pts_ref, cxt_ref, out_ref, *, N, TML):
    P = pts_ref[0]
    C = cxt_ref[0]
    pn2 = jnp.sum(P * P, axis=1, keepdims=True)
    cn2 = jnp.sum(C * C, axis=0, keepdims=True)
    dot = lax.dot_general(P, C, (((1,), (0,)), ((), ())),
                          preferred_element_type=F32,
                          precision=lax.Precision.HIGHEST)
    d = pn2 + cn2 - 2.0 * dot
    dc = jnp.min(d.reshape(N // _CH, _CH, TML), axis=1)
    out_ref[0] = dc


def _dc_chunkmin(xyz, pxt, M, N, TML=128):
    B = xyz.shape[0]
    NC = N // _CH
    body = functools.partial(_dc_body, N=N, TML=TML)
    return pl.pallas_call(
        body,
        grid=(B, M // TML),
        in_specs=[
            pl.BlockSpec((1, N, 3), lambda bb, t: (bb, 0, 0)),
            pl.BlockSpec((1, 3, TML), lambda bb, t: (bb, 0, t)),
        ],
        out_specs=pl.BlockSpec((1, NC, TML), lambda bb, t: (bb, 0, t)),
        out_shape=jax.ShapeDtypeStruct((B, NC, M), F32),
    )(xyz, pxt)


def _sc_ballquery(dc_flat, xp, yp, zp, B, M, N, NS, K, radius):
    NC = N // _CH
    r2 = radius * radius
    r2m = r2 * 1.0001 + 1e-5
    NW = 32
    RPW = (B * M) // NW
    assert (B * M) % NW == 0 and M % RPW == 0
    GMAX = NC // 16
    CAP = N + 16
    BIGF = jnp.float32(1e30)
    BIGI = jnp.int32(2 ** 30)

    mesh = plsc.VectorSubcoreMesh(core_axis_name="c", subcore_axis_name="s")

    @functools.partial(
        pl.kernel,
        mesh=mesh,
        out_type=jax.ShapeDtypeStruct((B * M * K,), jnp.int32),
        compiler_params=pltpu.CompilerParams(needs_layout_passes=False),
        scratch_types=[
            pltpu.VMEM((RPW * NC,), F32),
            pltpu.VMEM((N,), F32),
            pltpu.VMEM((N,), F32),
            pltpu.VMEM((N,), F32),
            pltpu.VMEM((2 * (NC + 16),), jnp.int32),
            pltpu.VMEM((2 * CAP,), jnp.int32),
            pltpu.VMEM((2 * CAP,), F32),
            pltpu.VMEM((RPW * K + 16,), jnp.int32),
            pltpu.SemaphoreType.DMA,
        ],
    )
    def kq(dc_hbm, xp_hbm, yp_hbm, zp_hbm, out_hbm,
           dcv, xv, yv, zv, ccv, piv, pdv, outv, sem):
        wid = lax.axis_index("s") * 2 + lax.axis_index("c")
        row0 = wid * RPW
        b = row0 // M
        cbase = row0 - b * M
        pltpu.sync_copy(dc_hbm.at[pl.ds(row0 * NC, RPW * NC)], dcv)
        pltpu.sync_copy(xp_hbm.at[pl.ds(b * NS, N)], xv)
        pltpu.sync_copy(yp_hbm.at[pl.ds(b * NS, N)], yv)
        pltpu.sync_copy(zp_hbm.at[pl.ds(b * NS, N)], zv)
        lanes = lax.broadcasted_iota(jnp.int32, (16,), 0)

        @plsc.parallel_loop(0, RPW, 1, unroll=2)
        def row_body(r):
            par = r % 2
            coff = par * (NC + 16)
            poff = par * CAP
            cloc = cbase + r
            csp = jnp.full((16,), cloc, jnp.int32)
            cx = plsc.load_gather(xv, [csp])
            cy = plsc.load_gather(yv, [csp])
            cz = plsc.load_gather(zv, [csp])

            def p1(c, carry):
                ncc_s, vm, im = carry
                v = dcv[pl.ds(r * NC + c * 16, 16)]
                ids = lanes + c * 16
                mask = v <= r2m
                cnt = plsc.all_reduce_population_count(mask)
                plsc.store_compressed(ccv.at[pl.ds(coff + ncc_s, 16)], ids,
                                      mask=mask)
                im = jnp.where(v < vm, ids, im)
                vm = jnp.minimum(vm, v)
                return ncc_s + cnt[0], vm, im

            ncc, vm, im = lax.fori_loop(
                0, GMAX, p1,
                (jnp.int32(0), jnp.full((16,), BIGF),
                 jnp.zeros((16,), jnp.int32)))
            mch = jnp.min(vm)
            amc = jnp.min(jnp.where(vm == mch, im, BIGI))
            plsc.store_scatter(ccv, [coff + lanes],
                               jnp.full((16,), amc, jnp.int32),
                               mask=(lanes == 0) & (ncc == 0))
            ncc = jnp.maximum(ncc, 1)

            def p2(g, carry):
                np_s, tmv, tav = carry
                gids = plsc.load_gather(ccv, [coff + lanes + g * 16])
                valid = (lanes + g * 16) < ncc
                gids = jnp.where(valid, gids, 0)
                for j in range(_CH):
                    pt = gids * _CH + j
                    dx = plsc.load_gather(xv, [pt]) - cx
                    dy = plsc.load_gather(yv, [pt]) - cy
                    dz = plsc.load_gather(zv, [pt]) - cz
                    d2 = dx * dx + dy * dy + dz * dz
                    d2 = jnp.where(valid, d2, BIGF)
                    inr = d2 <= r2
                    plsc.store_compressed(piv.at[pl.ds(poff + np_s, 16)], pt,
                                          mask=inr)
                    plsc.store_compressed(pdv.at[pl.ds(poff + np_s, 16)], d2,
                                          mask=inr)
                    np_s = np_s + plsc.all_reduce_population_count(inr)[0]
                    tav = jnp.where(d2 < tmv, pt, tav)
                    tmv = jnp.minimum(tmv, d2)
                return np_s, tmv, tav

            np_, tmv, tav = lax.fori_loop(
                0, (ncc + 15) // 16, p2,
                (jnp.int32(0), jnp.full((16,), BIGF),
                 jnp.zeros((16,), jnp.int32)))
            tmn = jnp.min(tmv)
            pad = jnp.min(jnp.where(tmv == tmn, tav, BIGI))
            padv = jnp.full((16,), pad, jnp.int32)

            off = r * K
            for s in range(0, K, 16):
                plsc.store_scatter(outv, [off + s + lanes], padv, mask=lanes < (K - s))

            @pl.when(np_ <= K)
            def _small():
                def emit(g, _c):
                    pos = g * 16 + lanes
                    vals = plsc.load_gather(piv, [poff + jnp.where(pos < np_, pos, 0)])
                    plsc.store_scatter(outv, [off + pos], vals, mask=pos < np_)
                    return _c
                lax.fori_loop(0, (np_ + 15) // 16, emit, 0)

            @pl.when(np_ > K)
            def _big():
                def ldrun(g):
                    pos = g * 16 + lanes
                    ok = pos < np_
                    posc = poff + jnp.where(ok, pos, 0)
                    kr = jnp.where(ok, plsc.load_gather(pdv, [posc]), BIGF)
                    vr = plsc.load_gather(piv, [posc])
                    return plsc.sort_key_val(kr, vr)

                def msplit(ak, av, bk, bv):
                    rbk = lax.rev(bk, (0,))
                    rbv = lax.rev(bv, (0,))
                    sel = ak <= rbk
                    lk = jnp.where(sel, ak, rbk)
                    lv = jnp.where(sel, av, rbv)
                    hk = jnp.where(sel, rbk, ak)
                    hv = jnp.where(sel, rbv, av)
                    lk, lv = plsc.sort_key_val(lk, lv)
                    hk, hv = plsc.sort_key_val(hk, hv)
                    return lk, lv, hk, hv

                s0k, s0v = ldrun(0)
                s1k, s1v = ldrun(1)
                a = msplit(s0k, s0v, s1k, s1v)

                def inc(g, carry):
                    a0k, a0v, a1k, a1v = carry
                    bk, bv = ldrun(g)
                    a1k, a1v, _x, _y = msplit(a1k, a1v, bk, bv)
                    a0k, a0v, a1k, a1v = msplit(a0k, a0v, a1k, a1v)
                    return a0k, a0v, a1k, a1v

                a0k, a0v, a1k, a1v = lax.fori_loop(2, (np_ + 15) // 16, inc, a)
                plsc.store_scatter(outv, [off + lanes], a0v,
                                   mask=lanes < 16)
                plsc.store_scatter(outv, [off + 16 + lanes], a1v,
                                   mask=lanes < (K - 16))


        pltpu.sync_copy(outv.at[pl.ds(0, RPW * K)],
                        out_hbm.at[pl.ds(row0 * K, RPW * K)])

    return kq(dc_flat, xp, yp, zp)


def _ball_query(xyz, pxt, xp, yp, zp, M, N, K, radius):
    B, NS = xyz.shape[0], xyz.shape[1]
    dct = _dc_chunkmin(xyz[:, :N], pxt[:, :, :M], M, N)
    dc_flat = jnp.transpose(dct, (0, 2, 1)).reshape(-1)
    idx = _sc_ballquery(dc_flat, xp, yp, zp, B, M, N, NS, K, radius)
    return idx.reshape(B, M, K)


def _sc_gather(table, idx):
    R = idx.shape[0]
    H = table.shape[1]
    NW = 32
    per_w = R // NW
    CH = 128
    n_chunks = per_w // CH
    assert per_w % CH == 0 and R % NW == 0

    mesh = plsc.VectorSubcoreMesh(core_axis_name="c", subcore_axis_name="s")

    @functools.partial(
        pl.kernel,
        mesh=mesh,
        out_type=jax.ShapeDtypeStruct((R, H), F32),
        scratch_types=[
            pltpu.VMEM((CH,), jnp.int32),
            pltpu.VMEM((CH, H), F32),
            pltpu.SemaphoreType.DMA,
        ],
    )
    def k(table_hbm, idx_hbm, out_hbm, idx_v, rows_v, sem):
        wid = lax.axis_index("s") * 2 + lax.axis_index("c")
        base = wid * per_w

        def body(i, carry):
            off = pl.multiple_of(base + i * CH, CH)
            pltpu.sync_copy(idx_hbm.at[pl.ds(off, CH)], idx_v)
            pltpu.async_copy(table_hbm.at[idx_v], rows_v, sem).wait()
            pltpu.sync_copy(rows_v, out_hbm.at[pl.ds(off, CH)])
            return carry

        lax.fori_loop(0, n_chunks, body, 0)

    return k(table, idx)


def _flat_idx(idx, B, M, K, N, TM):
    T = M // TM
    x = idx + (jnp.arange(B, dtype=jnp.int32) * N)[:, None, None]
    x = x.reshape(B, T, TM, K).transpose(0, 1, 3, 2)
    return x.reshape(B * M * K)


def _sa_tail_body(g2_ref, cx_ref, cf_ref, Wr_ref, c0_ref, g0_ref, be0_ref,
                  W1_ref, b1_ref, g1_ref, be1_ref,
                  Wres_ref, bres_ref, gres_ref, beres_ref,
                  Wfn_ref, Wrn_ref,
                  feat_ref, un_ref, *, K, TM, H, inv_r):
    cx = cx_ref[0]
    V = c0_ref[...] - _mm(cx, Wr_ref[...])
    Vt = jnp.concatenate([V] * K, axis=0)
    h = _gelu(_ln(g2_ref[...][:, :H] + Vt, g0_ref[...], be0_ref[...]))
    h1 = _ln(_mm(h, W1_ref[...]) + b1_ref[...], g1_ref[...], be1_ref[...])
    mx = h1[0:TM, :]
    for k in range(1, K):
        mx = jnp.maximum(mx, h1[k * TM:(k + 1) * TM, :])
    res = _ln(_mm(cf_ref[0], Wres_ref[...]) + bres_ref[...],
              gres_ref[...], beres_ref[...])
    out = _gelu(mx + res)
    feat_ref[0] = out
    un_ref[0] = _mm(out, Wfn_ref[...]) + _mm(cx, Wrn_ref[...])


def _sa_tail(G2, cxyz, cfeat, w, wnext, M, K, H, C2, inv_r, TM):
    B = cxyz.shape[0]
    T = M // TM
    Hg = G2.shape[1]
    Hn = wnext['Wf'].shape[1]
    body = functools.partial(_sa_tail_body, K=K, TM=TM, H=H, inv_r=inv_r)
    full = lambda a: pl.BlockSpec(a.shape, lambda bb, t: tuple(0 for _ in a.shape))
    return pl.pallas_call(
        body,
        grid=(B, T),
        in_specs=[
            pl.BlockSpec((TM * K, Hg), lambda bb, t: (bb * T + t, 0)),
            pl.BlockSpec((1, TM, 3), lambda bb, t: (bb, t, 0)),
            pl.BlockSpec((1, TM, cfeat.shape[-1]), lambda bb, t: (bb, t, 0)),
            full(w['Wr']), full(w['c0']), full(w['g0']), full(w['be0']),
            full(w['W1']), full(w['b1']), full(w['g1']), full(w['be1']),
            full(w['Wres']), full(w['bres']), full(w['gres']), full(w['beres']),
            full(wnext['Wf']), full(wnext['Wr']),
        ],
        out_specs=[
            pl.BlockSpec((1, TM, C2), lambda bb, t: (bb, t, 0)),
            pl.BlockSpec((1, TM, Hn), lambda bb, t: (bb, t, 0)),
        ],
        out_shape=[
            jax.ShapeDtypeStruct((B, M, C2), F32),
            jax.ShapeDtypeStruct((B, M, Hn), F32),
        ],
    )(G2, cxyz, cfeat, w['Wr'], w['c0'], w['g0'], w['be0'],
      w['W1'], w['b1'], w['g1'], w['be1'],
      w['Wres'], w['bres'], w['gres'], w['beres'],
      wnext['Wf'], wnext['Wr'])


def _irb_tail_body(g2_ref, cx_ref, f_ref, Wr_ref, c0_ref, g0_ref, be0_ref,
                   W1_ref, b1_ref, g1_ref, be1_ref,
                   Wc0_ref, bc0_ref, gc0_ref, bec0_ref,
                   Wc1_ref, bc1_ref, gc1_ref, bec1_ref,
                   Wfn_ref, Wrn_ref,
                   feat_ref, un_ref, *, K, TM, inv_r):
    cx = cx_ref[0]
    V = c0_ref[...] - _mm(cx, Wr_ref[...])
    Vt = jnp.concatenate([V] * K, axis=0)
    h = _gelu(_ln(g2_ref[...] + Vt, g0_ref[...], be0_ref[...]))
    h1 = _ln(_mm(h, W1_ref[...]) + b1_ref[...], g1_ref[...], be1_ref[...])
    mx = h1[0:TM, :]
    for k in range(1, K):
        mx = jnp.maximum(mx, h1[k * TM:(k + 1) * TM, :])
    hc = _gelu(_ln(_mm(mx, Wc0_ref[...]) + bc0_ref[...],
                   gc0_ref[...], bec0_ref[...]))
    hc = _ln(_mm(hc, Wc1_ref[...]) + bc1_ref[...], gc1_ref[...], bec1_ref[...])
    out = _gelu(hc + f_ref[0])
    feat_ref[0] = out
    un_ref[0] = _mm(out, Wfn_ref[...]) + _mm(cx, Wrn_ref[...])


def _final_tail_body(g2_ref, cx_ref, f_ref, Wr_ref, c0_ref, g0_ref, be0_ref,
                     W1_ref, b1_ref, g1_ref, be1_ref,
                     Wc0_ref, bc0_ref, gc0_ref, bec0_ref,
                     Wc1_ref, bc1_ref, gc1_ref, bec1_ref,
                     freqs_ref, gpW_ref, gpb_ref, gpg_ref, gpbe_ref,
                     opW_ref, opb_ref, opg_ref, opbe_ref,
                     out_ref, *, K, TM, inv_r):
    cx = cx_ref[0]
    V = c0_ref[...] - _mm(cx, Wr_ref[...])
    Vt = jnp.concatenate([V] * K, axis=0)
    h = _gelu(_ln(g2_ref[...] + Vt, g0_ref[...], be0_ref[...]))
    h1 = _ln(_mm(h, W1_ref[...]) + b1_ref[...], g1_ref[...], be1_ref[...])
    mx = h1[0:TM, :]
    for k in range(1, K):
        mx = jnp.maximum(mx, h1[k * TM:(k + 1) * TM, :])
    hc = _gelu(_ln(_mm(mx, Wc0_ref[...]) + bc0_ref[...],
                   gc0_ref[...], bec0_ref[...]))
    hc = _ln(_mm(hc, Wc1_ref[...]) + bc1_ref[...], gc1_ref[...], bec1_ref[...])
    feat = _gelu(hc + f_ref[0])
    fr = freqs_ref[...]
    parts = []
    for c in range(3):
        ang = cx[:, c:c + 1] * fr
        parts.append(jnp.sin(ang))
        parts.append(jnp.cos(ang))
    pe = jnp.concatenate(parts, axis=1)
    g = _gelu(_ln(_mm(pe, gpW_ref[...]) + gpb_ref[...],
                  gpg_ref[...], gpbe_ref[...]))
    feat = feat + g
    out_ref[0] = _ln(_mm(feat, opW_ref[...]) + opb_ref[...],
                     opg_ref[...], opbe_ref[...])


def _irb_tail(G2, cxyz, feat, w, wnext, M, K, H, C2, inv_r, TM, final=None):
    B = cxyz.shape[0]
    T = M // TM
    full = lambda a: pl.BlockSpec(a.shape, lambda bb, t: tuple(0 for _ in a.shape))
    common_in = [
        pl.BlockSpec((TM * K, H), lambda bb, t: (bb * T + t, 0)),
        pl.BlockSpec((1, TM, 3), lambda bb, t: (bb, t, 0)),
        pl.BlockSpec((1, TM, C2), lambda bb, t: (bb, t, 0)),
        full(w['Wr']), full(w['c0']), full(w['g0']), full(w['be0']),
        full(w['W1']), full(w['b1']), full(w['g1']), full(w['be1']),
        full(w['Wc0']), full(w['bc0']), full(w['gc0']), full(w['bec0']),
        full(w['Wc1']), full(w['bc1']), full(w['gc1']), full(w['bec1']),
    ]
    common_args = (G2, cxyz, feat, w['Wr'], w['c0'], w['g0'], w['be0'],
                   w['W1'], w['b1'], w['g1'], w['be1'],
                   w['Wc0'], w['bc0'], w['gc0'], w['bec0'],
                   w['Wc1'], w['bc1'], w['gc1'], w['bec1'])
    if final is None:
        Hn = wnext['Wf'].shape[1]
        body = functools.partial(_irb_tail_body, K=K, TM=TM, inv_r=inv_r)
        return pl.pallas_call(
            body,
            grid=(B, T),
            in_specs=common_in + [full(wnext['Wf']), full(wnext['Wr'])],
            out_specs=[
                pl.BlockSpec((1, TM, C2), lambda bb, t: (bb, t, 0)),
                pl.BlockSpec((1, TM, Hn), lambda bb, t: (bb, t, 0)),
            ],
            out_shape=[
                jax.ShapeDtypeStruct((B, M, C2), F32),
                jax.ShapeDtypeStruct((B, M, Hn), F32),
            ],
        )(*common_args, wnext['Wf'], wnext['Wr'])
    body = functools.partial(_final_tail_body, K=K, TM=TM, inv_r=inv_r)
    return pl.pallas_call(
        body,
        grid=(B, T),
        in_specs=common_in + [full(final[n]) for n in
                              ('freqs', 'gpW', 'gpb', 'gpg', 'gpbe',
                               'opW', 'opb', 'opg', 'opbe')],
        out_specs=pl.BlockSpec((1, TM, 256), lambda bb, t: (bb, t, 0)),
        out_shape=jax.ShapeDtypeStruct((B, M, 256), F32),
    )(*common_args, *(final[n] for n in
                      ('freqs', 'gpW', 'gpb', 'gpg', 'gpbe',
                       'opW', 'opb', 'opg', 'opbe')))


def _row(v):
    return v.reshape(1, -1)


def _prep_grouped(p, w0_key, cin_feat, radius):
    r = max(radius, 1e-6)
    W0 = p[w0_key]['W']
    Wrel, Wfeat, Wpos = W0[:3], W0[3:3 + cin_feat], W0[3 + cin_feat:]
    Wr = Wrel + p['rpe_W'] @ Wpos
    c0 = p['rpe_b'] @ Wpos + p[w0_key]['b']
    return {'Wf': Wfeat, 'Wr': Wr / r, 'c0': _row(c0),
            'g0': _row(p[w0_key]['g']), 'be0': _row(p[w0_key]['be'])}


def kernel(pointcloud, params):
    B, N0, _ = pointcloud.shape
    N1, N2 = N0 // 2, N0 // 4
    K1, K2 = 24, 32
    R1, R2 = 0.08, 0.16

    xyz = pointcloud[..., :3]
    pxt = jnp.transpose(xyz, (0, 2, 1))

    p1 = params['s1_sa']
    pi1 = params['s1_irb0']
    p2 = params['s2_sa']
    pi2 = params['s2_irb0']

    w1 = _prep_grouped(p1, 'mlp0', 64, R1)
    wi1 = _prep_grouped(pi1, 'la0', 128, R1)
    w2 = _prep_grouped(p2, 'mlp0', 128, R2)
    wi2 = _prep_grouped(pi2, 'la0', 256, R2)

    def mlp_w(q, name):
        return {name + 'W': q['W'], name + 'b': _row(q['b']),
                name + 'g': _row(q['g']), name + 'be': _row(q['be'])}

    sa1_w = dict(w1, W1=p1['mlp1']['W'], b1=_row(p1['mlp1']['b']),
                 g1=_row(p1['mlp1']['g']), be1=_row(p1['mlp1']['be']),
                 Wres=p1['res']['W'], bres=_row(p1['res']['b']),
                 gres=_row(p1['res']['g']), beres=_row(p1['res']['be']))
    irb1_w = dict(wi1, W1=pi1['la1']['W'], b1=_row(pi1['la1']['b']),
                  g1=_row(pi1['la1']['g']), be1=_row(pi1['la1']['be']),
                  Wc0=pi1['cm0']['W'], bc0=_row(pi1['cm0']['b']),
                  gc0=_row(pi1['cm0']['g']), bec0=_row(pi1['cm0']['be']),
                  Wc1=pi1['cm1']['W'], bc1=_row(pi1['cm1']['b']),
                  gc1=_row(pi1['cm1']['g']), bec1=_row(pi1['cm1']['be']))
    sa2_w = dict(w2, W1=p2['mlp1']['W'], b1=_row(p2['mlp1']['b']),
                 g1=_row(p2['mlp1']['g']), be1=_row(p2['mlp1']['be']),
                 Wres=p2['res']['W'], bres=_row(p2['res']['b']),
                 gres=_row(p2['res']['g']), beres=_row(p2['res']['be']))
    irb2_w = dict(wi2, W1=pi2['la1']['W'], b1=_row(pi2['la1']['b']),
                  g1=_row(pi2['la1']['g']), be1=_row(pi2['la1']['be']),
                  Wc0=pi2['cm0']['W'], bc0=_row(pi2['cm0']['b']),
                  gc0=_row(pi2['cm0']['g']), bec0=_row(pi2['cm0']['be']),
                  Wc1=pi2['cm1']['W'], bc1=_row(pi2['cm1']['b']),
                  gc1=_row(pi2['cm1']['g']), bec1=_row(pi2['cm1']['be']))

    half = 16
    freqs = jnp.exp(-jnp.log(10000.0) *
                    jnp.arange(half, dtype=F32) / (half - 1)).reshape(1, half)
    final_w = {'freqs': freqs,
               'gpW': params['gp_W'], 'gpb': _row(params['gp_b']),
               'gpg': _row(params['gp_g']), 'gpbe': _row(params['gp_be']),
               'opW': params['op_W'], 'opb': _row(params['op_b']),
               'opg': _row(params['op_g']), 'opbe': _row(params['op_be'])}

    xp = xyz[..., 0].reshape(-1)
    yp = xyz[..., 1].reshape(-1)
    zp = xyz[..., 2].reshape(-1)
    idx1 = _ball_query(xyz, pxt, xp, yp, zp, N1, N0, K1, R1)
    idx2 = _ball_query(xyz, pxt, xp, yp, zp, N1, N1, K1, R1)
    idx3 = _ball_query(xyz, pxt, xp, yp, zp, N2, N1, K2, R2)
    idx4 = _ball_query(xyz, pxt, xp, yp, zp, N2, N2, K2, R2)

    TM1, TMi1, TM2, TMi2 = 256, 128, 128, 64
    fidx1 = _flat_idx(idx1, B, N1, K1, N0, TM1)
    fidx2 = _flat_idx(idx2, B, N1, K1, N1, TMi1)
    fidx3 = _flat_idx(idx3, B, N2, K2, N1, TM2)
    fidx4 = _flat_idx(idx4, B, N2, K2, N2, TMi2)

    pad128 = lambda a: jnp.pad(a, ((0, 0), (0, 128 - a.shape[1])))
    feat0, U1 = _head(pointcloud, params['s0_head']['W'],
                      _row(params['s0_head']['b']), _row(params['s0_head']['g']),
                      _row(params['s0_head']['be']),
                      pad128(w1['Wf']), pad128(w1['Wr']))

    G1 = _sc_gather(U1.reshape(B * N0, 128), fidx1)
    feat1, U2 = _sa_tail(G1, xyz, feat0, sa1_w, wi1, N1, K1, 64, 128,
                         1.0 / max(R1, 1e-6), TM1)

    G2 = _sc_gather(U2.reshape(B * N1, 256), fidx2)
    feat1b, U3 = _irb_tail(G2, xyz, feat1, irb1_w, w2, N1, K1, 256, 128,
                           1.0 / max(R1, 1e-6), TMi1)

    G3 = _sc_gather(U3.reshape(B * N1, 128), fidx3)
    feat2, U4 = _sa_tail(G3, xyz, feat1b, sa2_w, wi2, N2, K2, 128, 256,
                         1.0 / max(R2, 1e-6), TM2)

    G4 = _sc_gather(U4.reshape(B * N2, 512), fidx4)
    out = _irb_tail(G4, xyz, feat2, irb2_w, None, N2, K2, 512, 256,
                    1.0 / max(R2, 1e-6), TMi2, final=final_w)
    return out
